# Initial kernel scaffold; baseline (speedup 1.0000x reference)
import jax
import jax.numpy as jnp
from jax.experimental import pallas as pl


def kernel(output, adj):
    raise NotImplementedError("write your pallas kernel here")


# speedup vs baseline: 6.8561x; 6.8561x over previous
import functools

import jax
import jax.numpy as jnp
from jax import lax
from jax.experimental import pallas as pl
from jax.experimental.pallas import tpu as pltpu
from jax.experimental.pallas import tpu_sc as plsc

NC = 2
NS = 16
NW = NC * NS
L = 16
CHUNK = 16
NH = 7
RPC = CHUNK * NH


@functools.lru_cache(maxsize=None)
def _make_partial_kernel(nbatch: int, npw: int, d: int):
    nsteps = nbatch * npw // CHUNK
    giters = nsteps // 2
    awords = nbatch * npw * NH

    mesh = plsc.VectorSubcoreMesh(core_axis_name="c", subcore_axis_name="s")

    @functools.partial(
        pl.kernel,
        mesh=mesh,
        out_type=jax.ShapeDtypeStruct((NW, L), jnp.float32),
        scratch_types=[
            pltpu.VMEM((awords,), jnp.int32),
            pltpu.VMEM((RPC, d), jnp.float32),
            pltpu.VMEM((RPC, d), jnp.float32),
            pltpu.VMEM((L,), jnp.float32),
            pltpu.SemaphoreType.DMA,
            pltpu.SemaphoreType.DMA,
        ],
    )
    def nh_partial(table, adjw, out, adjv, rows0, rows1, accv, sem0, sem1):
        wid = lax.axis_index("s") * NC + lax.axis_index("c")
        pltpu.sync_copy(
            adjw.at[pl.ds(pl.multiple_of(wid * awords, 8), awords)], adjv)

        def copy(s, buf, sem):
            off = pl.multiple_of(s * RPC, 8)
            return pltpu.make_async_copy(
                table.at[adjv.at[pl.ds(off, RPC)]], buf, sem)

        copy(0, rows0, sem0).start()
        copy(1, rows1, sem1).start()

        def chunk(buf, accs):
            def node(i, accs):
                base = i * NH
                nxt = []
                for j in range(d // L):
                    c = buf[base, pl.ds(j * L, L)]
                    a = accs[j]
                    for k in range(1, NH):
                        a = a + jnp.abs(c - buf[base + k, pl.ds(j * L, L)])
                    nxt.append(a)
                return tuple(nxt)
            return lax.fori_loop(0, CHUNK, node, accs)

        def gstep(g, accs):
            s0 = 2 * g
            copy(s0, rows0, sem0).wait()
            accs = chunk(rows0, accs)

            @pl.when(s0 + 2 < nsteps)
            def _():
                copy(s0 + 2, rows0, sem0).start()

            copy(s0 + 1, rows1, sem1).wait()
            accs = chunk(rows1, accs)

            @pl.when(s0 + 3 < nsteps)
            def _():
                copy(s0 + 3, rows1, sem1).start()

            return accs

        accs = tuple(jnp.zeros((L,), jnp.float32) for _ in range(d // L))
        accs = lax.fori_loop(0, giters, gstep, accs)
        total = accs[0]
        for a in accs[1:]:
            total = total + a
        accv[...] = total
        pltpu.sync_copy(accv, out.at[wid])

    return nh_partial


def kernel(output, adj):
    nbatch, n, d = output.shape
    nh = adj.shape[1]
    assert nh == NH and d % L == 0
    npw = -(-n // (NW * CHUNK)) * CHUNK
    npad = NW * npw
    adj_pad = jnp.concatenate(
        [adj, jnp.zeros((npad - n, nh), jnp.int32)], axis=0)
    adj_w = adj_pad.reshape(NW, npw, nh)
    adj_b = jnp.stack([adj_w + b * n for b in range(nbatch)], axis=1)
    adj_flat = adj_b.reshape(-1)
    table = output.reshape(nbatch * n, d)
    parts = _make_partial_kernel(nbatch, npw, d)(table, adj_flat)
    denom = nbatch * n * (nh - 1) * d
    return jnp.sqrt(jnp.sum(parts) / jnp.float32(denom))
